# baseline (speedup 1.0000x reference)
import functools

import jax
import jax.numpy as jnp
from jax import lax
from jax.experimental import pallas as pl
from jax.experimental.pallas import tpu as pltpu
from jax.experimental.pallas import tpu_sc as plsc

_CHUNK = 800


@functools.lru_cache(maxsize=None)
def _make_relayout(vocab: int, dim: int):
    info = plsc.get_sparse_core_info()
    nc, ns = info.num_cores, info.num_subcores
    nw = nc * ns
    lanes_per_blk = 128
    nblk_full = vocab // lanes_per_blk
    tail = vocab - nblk_full * lanes_per_blk
    pairs_per_blk = lanes_per_blk * dim // 128
    mesh = plsc.VectorSubcoreMesh(core_axis_name="c", subcore_axis_name="s")

    @functools.partial(
        pl.kernel,
        out_type=jax.ShapeDtypeStruct((vocab * dim // 128, 128), jnp.float32),
        mesh=mesh,
        scratch_types=[
            pltpu.VMEM((dim, 128), jnp.float32),
            pltpu.VMEM((pairs_per_blk, 128), jnp.float32),
        ],
        compiler_params=pltpu.CompilerParams(
            use_tc_tiling_on_sc=True,
            needs_layout_passes=False,
            disable_bounds_checks=True,
        ),
    )
    def relayout(tt_hbm, packed_hbm, v_in, v_out):
        wid = lax.axis_index("s") * nc + lax.axis_index("c")
        n_mine = nblk_full // nw + jnp.where(wid < nblk_full % nw, 1, 0)

        def do_block(i, carry):
            blk = i * nw + wid
            v0 = pl.multiple_of(blk * lanes_per_blk, lanes_per_blk)
            pltpu.sync_copy(tt_hbm.at[:, pl.ds(v0, lanes_per_blk)], v_in)
            def do_row(r, carry2):
                for g in range(8):
                    d_vec = lax.iota(jnp.int32, 16) + (g % 4) * 16
                    c_vec = jnp.full((16,), 2 * r + g // 4, jnp.int32)
                    vals = plsc.load_gather(v_in, [d_vec, c_vec])
                    v_out[r, pl.ds(g * 16, 16)] = vals
                return carry2

            lax.fori_loop(0, pairs_per_blk, do_row, 0, unroll=4)
            pltpu.sync_copy(
                v_out,
                packed_hbm.at[
                    pl.ds(pl.multiple_of(blk * pairs_per_blk, 8),
                          pairs_per_blk)
                ],
            )
            return carry

        lax.fori_loop(0, n_mine, do_block, 0)

        if tail:
            tail_pairs = tail * dim // 128

            @pl.when(wid == nw - 1)
            def _():
                v0 = pl.multiple_of(
                    jnp.int32(nblk_full) * lanes_per_blk, lanes_per_blk
                )
                pltpu.sync_copy(tt_hbm.at[:, pl.ds(v0, lanes_per_blk)], v_in)

                def do_row(r, carry2):
                    for g in range(8):
                        d_vec = lax.iota(jnp.int32, 16) + (g % 4) * 16
                        c_vec = jnp.full((16,), 2 * r + g // 4, jnp.int32)
                        vals = plsc.load_gather(v_in, [d_vec, c_vec])
                        v_out[r, pl.ds(g * 16, 16)] = vals
                    return carry2

                lax.fori_loop(0, tail_pairs, do_row, 0, unroll=4)
                pltpu.sync_copy(
                    v_out.at[pl.ds(0, tail_pairs)],
                    packed_hbm.at[pl.ds(nblk_full * pairs_per_blk, tail_pairs)],
                )

    return relayout


@functools.lru_cache(maxsize=None)
def _make_lookup(num_idx: int, vocab: int, dim: int):
    info = plsc.get_sparse_core_info()
    nc, ns = info.num_cores, info.num_subcores
    nw = nc * ns
    assert num_idx % (nw * _CHUNK) == 0
    b_per_w = num_idx // nw
    nchunks = b_per_w // _CHUNK
    mesh = plsc.VectorSubcoreMesh(core_axis_name="c", subcore_axis_name="s")
    assert nchunks % 2 == 0 and nchunks >= 4

    @functools.partial(
        pl.kernel,
        out_type=jax.ShapeDtypeStruct((num_idx, dim), jnp.float32),
        mesh=mesh,
        scratch_types=[
            pltpu.VMEM((b_per_w,), jnp.int32),
            pltpu.VMEM((_CHUNK, dim), jnp.float32),
            pltpu.VMEM((_CHUNK, dim), jnp.float32),
            pltpu.SemaphoreType.DMA,
            pltpu.SemaphoreType.DMA,
            pltpu.SemaphoreType.DMA,
            pltpu.SemaphoreType.DMA,
        ],
        compiler_params=pltpu.CompilerParams(use_tc_tiling_on_sc=False),
    )
    def lookup(word_hbm, table_hbm, out_hbm, idx_v, rows0, rows1,
               g0, g1, o0, o1):
        wid = lax.axis_index("s") * nc + lax.axis_index("c")
        base = wid * b_per_w
        pltpu.sync_copy(word_hbm.at[pl.ds(base, b_per_w)], idx_v)
        bufs = (rows0, rows1)
        gsems = (g0, g1)
        osems = (o0, o1)

        def gather(c, buf, sem):
            return pltpu.make_async_copy(
                table_hbm.at[idx_v.at[pl.ds(c * _CHUNK, _CHUNK)]], buf, sem
            )

        def writeback(c, buf, sem):
            return pltpu.make_async_copy(
                buf, out_hbm.at[pl.ds(base + c * _CHUNK, _CHUNK)], sem
            )

        gather(0, rows0, g0).start()
        gather(1, rows1, g1).start()

        def pair(p, carry):
            for b in range(2):
                c = 2 * p + b
                gather(c, bufs[b], gsems[b]).wait()
                writeback(c, bufs[b], osems[b]).start()
                writeback(c, bufs[b], osems[b]).wait()
                gather(c + 2, bufs[b], gsems[b]).start()
            return carry

        lax.fori_loop(0, nchunks // 2 - 1, pair, 0)

        for b in range(2):
            c = nchunks - 2 + b
            gather(c, bufs[b], gsems[b]).wait()
            writeback(c, bufs[b], osems[b]).start()
        for b in range(2):
            c = nchunks - 2 + b
            writeback(c, bufs[b], osems[b]).wait()

    return lookup


def kernel(word, table):
    batch, hist = word.shape
    vocab, dim = table.shape
    packed = _make_relayout(vocab, dim)(table.T)
    flat = word.reshape(batch * hist)
    out = _make_lookup(batch * hist, vocab, dim)(
        flat, packed.reshape(vocab, dim)
    )
    return out.reshape(batch, hist, dim)


# speedup vs baseline: 1.1245x; 1.1245x over previous
import functools

import jax
import jax.numpy as jnp
from jax import lax
from jax.experimental import pallas as pl
from jax.experimental.pallas import tpu as pltpu
from jax.experimental.pallas import tpu_sc as plsc

_CHUNK = 800


@functools.lru_cache(maxsize=None)
def _make_relayout(vocab: int, dim: int):
    info = plsc.get_sparse_core_info()
    nc, ns = info.num_cores, info.num_subcores
    nw = nc * ns
    lanes_per_blk = 128
    nblk_full = vocab // lanes_per_blk
    tail = vocab - nblk_full * lanes_per_blk
    pairs_per_blk = lanes_per_blk * dim // 128
    mesh = plsc.VectorSubcoreMesh(core_axis_name="c", subcore_axis_name="s")

    blk_elems = pairs_per_blk * 128

    @functools.partial(
        pl.kernel,
        out_type=jax.ShapeDtypeStruct((vocab * dim,), jnp.float32),
        mesh=mesh,
        scratch_types=[
            pltpu.VMEM((dim, 128), jnp.float32),
            pltpu.VMEM((blk_elems,), jnp.float32),
        ],
        compiler_params=pltpu.CompilerParams(
            use_tc_tiling_on_sc=True,
            needs_layout_passes=False,
            disable_bounds_checks=True,
        ),
    )
    def relayout(tt_hbm, packed_hbm, v_in, v_out):
        wid = lax.axis_index("s") * nc + lax.axis_index("c")
        n_mine = nblk_full // nw + jnp.where(wid < nblk_full % nw, 1, 0)

        lane = lax.iota(jnp.int32, 16)
        p_vec = (lane // 2) * 128 + (lane % 2) * 64

        def transpose_block(ngroups):
            for g in range(ngroups):
                def col(d, idx):
                    vals = v_in[d, pl.ds(g * 16, 16)]
                    plsc.store_scatter(v_out, [idx], vals)
                    return idx + 1

                lax.fori_loop(0, dim, col, p_vec + g * 1024, unroll=8)

        def do_block(i, carry):
            blk = i * nw + wid
            v0 = pl.multiple_of(blk * lanes_per_blk, lanes_per_blk)
            pltpu.sync_copy(tt_hbm.at[:, pl.ds(v0, lanes_per_blk)], v_in)
            transpose_block(8)
            pltpu.sync_copy(
                v_out,
                packed_hbm.at[
                    pl.ds(pl.multiple_of(blk * blk_elems, 8), blk_elems)
                ],
            )
            return carry

        lax.fori_loop(0, n_mine, do_block, 0)

        if tail:
            tail_elems = tail * dim

            @pl.when(wid == nw - 1)
            def _():
                v0 = pl.multiple_of(
                    jnp.int32(nblk_full) * lanes_per_blk, lanes_per_blk
                )
                pltpu.sync_copy(tt_hbm.at[:, pl.ds(v0, lanes_per_blk)], v_in)
                transpose_block(tail // 16)
                pltpu.sync_copy(
                    v_out.at[pl.ds(0, tail_elems)],
                    packed_hbm.at[pl.ds(nblk_full * blk_elems, tail_elems)],
                )

    return relayout


@functools.lru_cache(maxsize=None)
def _make_lookup(num_idx: int, vocab: int, dim: int):
    info = plsc.get_sparse_core_info()
    nc, ns = info.num_cores, info.num_subcores
    nw = nc * ns
    assert num_idx % (nw * _CHUNK) == 0
    b_per_w = num_idx // nw
    nchunks = b_per_w // _CHUNK
    mesh = plsc.VectorSubcoreMesh(core_axis_name="c", subcore_axis_name="s")
    assert nchunks % 2 == 0 and nchunks >= 4

    @functools.partial(
        pl.kernel,
        out_type=jax.ShapeDtypeStruct((num_idx, dim), jnp.float32),
        mesh=mesh,
        scratch_types=[
            pltpu.VMEM((b_per_w,), jnp.int32),
            pltpu.VMEM((_CHUNK, dim), jnp.float32),
            pltpu.VMEM((_CHUNK, dim), jnp.float32),
            pltpu.SemaphoreType.DMA,
            pltpu.SemaphoreType.DMA,
            pltpu.SemaphoreType.DMA,
            pltpu.SemaphoreType.DMA,
        ],
        compiler_params=pltpu.CompilerParams(use_tc_tiling_on_sc=False),
    )
    def lookup(word_hbm, table_hbm, out_hbm, idx_v, rows0, rows1,
               g0, g1, o0, o1):
        wid = lax.axis_index("s") * nc + lax.axis_index("c")
        base = wid * b_per_w
        pltpu.sync_copy(word_hbm.at[pl.ds(base, b_per_w)], idx_v)
        bufs = (rows0, rows1)
        gsems = (g0, g1)
        osems = (o0, o1)

        def gather(c, buf, sem):
            return pltpu.make_async_copy(
                table_hbm.at[idx_v.at[pl.ds(c * _CHUNK, _CHUNK)]], buf, sem
            )

        def writeback(c, buf, sem):
            return pltpu.make_async_copy(
                buf, out_hbm.at[pl.ds(base + c * _CHUNK, _CHUNK)], sem
            )

        gather(0, rows0, g0).start()
        gather(1, rows1, g1).start()

        def pair(p, carry):
            for b in range(2):
                c = 2 * p + b
                gather(c, bufs[b], gsems[b]).wait()
                writeback(c, bufs[b], osems[b]).start()
                writeback(c, bufs[b], osems[b]).wait()
                gather(c + 2, bufs[b], gsems[b]).start()
            return carry

        lax.fori_loop(0, nchunks // 2 - 1, pair, 0)

        for b in range(2):
            c = nchunks - 2 + b
            gather(c, bufs[b], gsems[b]).wait()
            writeback(c, bufs[b], osems[b]).start()
        for b in range(2):
            c = nchunks - 2 + b
            writeback(c, bufs[b], osems[b]).wait()

    return lookup


def kernel(word, table):
    batch, hist = word.shape
    vocab, dim = table.shape
    packed = _make_relayout(vocab, dim)(table.T)
    flat = word.reshape(batch * hist)
    out = _make_lookup(batch * hist, vocab, dim)(
        flat, packed.reshape(vocab, dim)
    )
    return out.reshape(batch, hist, dim)


# speedup vs baseline: 1.9915x; 1.7711x over previous
import functools

import jax
import jax.numpy as jnp
from jax import lax
from jax.experimental import pallas as pl
from jax.experimental.pallas import tpu as pltpu
from jax.experimental.pallas import tpu_sc as plsc

_CHUNK = 800


@functools.lru_cache(maxsize=None)
def _make_lookup(num_idx: int, vocab: int, dim: int):
    info = plsc.get_sparse_core_info()
    nc, ns = info.num_cores, info.num_subcores
    nw = nc * ns
    assert num_idx % (nw * _CHUNK) == 0
    b_per_w = num_idx // nw
    nchunks = b_per_w // _CHUNK
    mesh = plsc.VectorSubcoreMesh(core_axis_name="c", subcore_axis_name="s")

    assert nchunks % 2 == 0 and nchunks >= 4

    @functools.partial(
        pl.kernel,
        out_type=jax.ShapeDtypeStruct((num_idx, dim), jnp.float32),
        mesh=mesh,
        scratch_types=[
            pltpu.VMEM((b_per_w,), jnp.int32),
            pltpu.VMEM((_CHUNK, dim), jnp.float32),
            pltpu.VMEM((_CHUNK, dim), jnp.float32),
            pltpu.SemaphoreType.DMA,
            pltpu.SemaphoreType.DMA,
            pltpu.SemaphoreType.DMA,
            pltpu.SemaphoreType.DMA,
        ],
        compiler_params=pltpu.CompilerParams(use_tc_tiling_on_sc=False),
    )
    def lookup(word_hbm, table_hbm, out_hbm, idx_v, rows0, rows1,
               g0, g1, o0, o1):
        wid = lax.axis_index("s") * nc + lax.axis_index("c")
        base = wid * b_per_w
        pltpu.sync_copy(word_hbm.at[pl.ds(base, b_per_w)], idx_v)
        bufs = (rows0, rows1)
        gsems = (g0, g1)
        osems = (o0, o1)

        def gather(c, buf, sem):
            return pltpu.make_async_copy(
                table_hbm.at[idx_v.at[pl.ds(c * _CHUNK, _CHUNK)]], buf, sem
            )

        def writeback(c, buf, sem):
            return pltpu.make_async_copy(
                buf, out_hbm.at[pl.ds(base + c * _CHUNK, _CHUNK)], sem
            )

        gather(0, rows0, g0).start()
        gather(1, rows1, g1).start()

        def pair(p, carry):
            for b in range(2):
                c = 2 * p + b
                gather(c, bufs[b], gsems[b]).wait()
                writeback(c, bufs[b], osems[b]).start()
                writeback(c, bufs[b], osems[b]).wait()
                gather(c + 2, bufs[b], gsems[b]).start()
            return carry

        lax.fori_loop(0, nchunks // 2 - 1, pair, 0)

        for b in range(2):
            c = nchunks - 2 + b
            gather(c, bufs[b], gsems[b]).wait()
            writeback(c, bufs[b], osems[b]).start()
        for b in range(2):
            c = nchunks - 2 + b
            writeback(c, bufs[b], osems[b]).wait()

    return lookup


def kernel(word, table):
    batch, hist = word.shape
    vocab, dim = table.shape
    flat = word.reshape(batch * hist)
    out = _make_lookup(batch * hist, vocab, dim)(flat, table)
    return out.reshape(batch, hist, dim)
